# baseline (speedup 1.0000x reference)
import dataclasses
import functools

import jax
import jax.numpy as jnp
from jax import lax
from jax.experimental import pallas as pl
from jax.experimental.pallas import tpu as pltpu
from jax.experimental.pallas import tpu_sc as plsc

N = 10000
E = 320000
NC, NS = 2, 16
NW = NC * NS
EPW = E // NW
CHUNK = 2000
L = 16

_mesh = plsc.VectorSubcoreMesh(
    core_axis_name="c", subcore_axis_name="s", num_cores=NC, num_subcores=NS)

_cp = pltpu.CompilerParams()
if "needs_layout_passes" in pltpu.CompilerParams.__dataclass_fields__:
    _cp = dataclasses.replace(_cp, needs_layout_passes=False)


def _wid():
    return lax.axis_index("c") * NS + lax.axis_index("s")


RED = 640
NPAD = NS * RED


def _rsqrt_newton(x):
    i = plsc.bitcast(x, jnp.int32)
    seed = jnp.full((L,), 0x5F3759DF, jnp.int32) - lax.shift_right_logical(
        i, jnp.ones((L,), jnp.int32))
    y = plsc.bitcast(seed, jnp.float32)
    for _ in range(3):
        y = y * (1.5 - 0.5 * x * y * y)
    return y


@functools.partial(
    pl.kernel,
    out_type=(jax.ShapeDtypeStruct((NW, 16, N), jnp.float32),
              jax.ShapeDtypeStruct((E,), jnp.float32),
              jax.ShapeDtypeStruct((NC * NPAD,), jnp.float32),
              jax.ShapeDtypeStruct((NC * NS * N,), jnp.float32)),
    mesh=_mesh,
    compiler_params=_cp,
    name="sc_layer1",
    scratch_types=[
        pltpu.VMEM((N,), jnp.float32),
        pltpu.VMEM((EPW,), jnp.float32),
        pltpu.VMEM((EPW,), jnp.int32),
        pltpu.VMEM((EPW,), jnp.int32),
        pltpu.VMEM((NS * (RED // 2),), jnp.float32),
        pltpu.SemaphoreType.DMA,
    ] + [pltpu.VMEM((N,), jnp.float32)] * 8,
)
def _sc_layer1(eix_hbm, ew_hbm, hT_hbm,
               out_hbm, norm_hbm, dis_hbm, degs_hbm,
               dis_v, nrm_v, src_v, dst_v, red_blk, dma_sem,
               *rows):
    h_fs = rows[:4]
    acc_fs = rows[4:]
    sid = lax.axis_index("s")
    wid = _wid()
    z16 = jnp.zeros((L,), jnp.float32)

    @plsc.parallel_loop(0, N, step=L, unroll=8)
    def _(i):
        dis_v[pl.ds(i, L)] = z16

    for half in range(2):
        dbase = sid * (2 * EPW) + half * EPW
        pltpu.sync_copy(eix_hbm.at[pl.ds(E + dbase, EPW)], src_v)
        pltpu.sync_copy(ew_hbm.at[pl.ds(dbase, EPW)], nrm_v)

        @plsc.parallel_loop(0, EPW, step=L, unroll=8)
        def _(g):
            plsc.addupdate_scatter(
                dis_v, [src_v[pl.ds(g, L)]], nrm_v[pl.ds(g, L)])

    cid = lax.axis_index("c")
    pltpu.sync_copy(dis_v, degs_hbm.at[pl.ds((cid * NS + sid) * N, N)])
    plsc.subcore_barrier()
    tstart = sid * RED
    hw = RED // 2
    for half in range(2):
        copies = [
            pltpu.async_copy(
                degs_hbm.at[pl.ds((cid * NS + j) * N + tstart + half * hw, hw)],
                red_blk.at[pl.ds(j * hw, hw)], dma_sem)
            for j in range(NS)
        ]
        for c in copies:
            c.wait()

        @plsc.parallel_loop(0, hw, step=L, unroll=2)
        def _(k):
            acc = red_blk[pl.ds(k, L)]
            for j in range(1, NS):
                acc = acc + red_blk[pl.ds(j * hw + k, L)]
            dis_v[pl.ds(half * hw + k, L)] = _rsqrt_newton(acc + 1.0)

    pltpu.sync_copy(dis_v.at[pl.ds(0, RED)],
                    dis_hbm.at[pl.ds(cid * NPAD + tstart, RED)])
    plsc.subcore_barrier()
    pltpu.sync_copy(dis_hbm.at[pl.ds(cid * NPAD, N)], dis_v)

    base = wid * EPW
    pltpu.sync_copy(eix_hbm.at[pl.ds(base, EPW)], src_v)
    pltpu.sync_copy(eix_hbm.at[pl.ds(E + base, EPW)], dst_v)
    pltpu.sync_copy(ew_hbm.at[pl.ds(base, EPW)], nrm_v)

    @plsc.parallel_loop(0, EPW, step=L, unroll=8)
    def _(g):
        dsv = plsc.load_gather(dis_v, [src_v[pl.ds(g, L)]])
        ddv = plsc.load_gather(dis_v, [dst_v[pl.ds(g, L)]])
        nrm_v[pl.ds(g, L)] = dsv * nrm_v[pl.ds(g, L)] * ddv

    pltpu.sync_copy(nrm_v, norm_hbm.at[pl.ds(base, EPW)])

    for p in range(4):
        for f in range(4):
            pltpu.sync_copy(hT_hbm.at[p * 4 + f], h_fs[f])

        @plsc.parallel_loop(0, N, step=L, unroll=8)
        def _(i):
            for f in range(4):
                acc_fs[f][pl.ds(i, L)] = z16

        @plsc.parallel_loop(0, EPW, step=L, unroll=16)
        def _(g):
            s16 = src_v[pl.ds(g, L)]
            d16 = dst_v[pl.ds(g, L)]
            n16 = nrm_v[pl.ds(g, L)]
            for f in range(4):
                hf = plsc.load_gather(h_fs[f], [s16])
                plsc.addupdate_scatter(acc_fs[f], [d16], hf * n16)

        for f in range(4):
            pltpu.sync_copy(acc_fs[f], out_hbm.at[wid, p * 4 + f])


def _make_sc_agg(F, G, make_norm):
    n_pass = F // G
    out_types = [jax.ShapeDtypeStruct((NW, F, N), jnp.float32)]
    scratch = [
        pltpu.VMEM((EPW,), jnp.float32),
        pltpu.VMEM((EPW,), jnp.int32),
        pltpu.VMEM((EPW,), jnp.int32),
    ] + [pltpu.VMEM((N,), jnp.float32)] * (2 * G)
    if make_norm:
        out_types.append(jax.ShapeDtypeStruct((E,), jnp.float32))
        scratch.append(pltpu.VMEM((N,), jnp.float32))

    @functools.partial(
        pl.kernel,
        out_type=tuple(out_types) if len(out_types) > 1 else out_types[0],
        mesh=_mesh,
        compiler_params=_cp,
        name=f"sc_agg{F}",
        scratch_types=scratch,
    )
    def _sc_agg(eix_hbm, ewn_hbm, hT_hbm, *rest):
        if make_norm:
            dis_hbm, out_hbm, norm_hbm, nrm_v, src_v, dst_v, *rows = rest
            dis_v = rows[-1]
            rows = rows[:-1]
        else:
            out_hbm, nrm_v, src_v, dst_v, *rows = rest
        h_fs = rows[:G]
        acc_fs = rows[G:]
        base = _wid() * EPW
        pltpu.sync_copy(eix_hbm.at[pl.ds(base, EPW)], src_v)
        pltpu.sync_copy(eix_hbm.at[pl.ds(E + base, EPW)], dst_v)
        pltpu.sync_copy(ewn_hbm.at[pl.ds(base, EPW)], nrm_v)

        if make_norm:
            pltpu.sync_copy(dis_hbm, dis_v)

            @plsc.parallel_loop(0, EPW, step=L, unroll=8)
            def _(g):
                s16 = src_v[pl.ds(g, L)]
                d16 = dst_v[pl.ds(g, L)]
                dsv = plsc.load_gather(dis_v, [s16])
                ddv = plsc.load_gather(dis_v, [d16])
                nrm_v[pl.ds(g, L)] = dsv * nrm_v[pl.ds(g, L)] * ddv

            pltpu.sync_copy(nrm_v, norm_hbm.at[pl.ds(base, EPW)])

        z16 = jnp.zeros((L,), jnp.float32)
        for p in range(n_pass):
            for f in range(G):
                pltpu.sync_copy(hT_hbm.at[p * G + f], h_fs[f])

            @plsc.parallel_loop(0, N, step=L, unroll=8)
            def _(i):
                for f in range(G):
                    acc_fs[f][pl.ds(i, L)] = z16

            @plsc.parallel_loop(0, EPW, step=L, unroll=8)
            def _(g):
                s16 = src_v[pl.ds(g, L)]
                d16 = dst_v[pl.ds(g, L)]
                n16 = nrm_v[pl.ds(g, L)]
                for f in range(G):
                    hf = plsc.load_gather(h_fs[f], [s16])
                    plsc.addupdate_scatter(acc_fs[f], [d16], hf * n16)

            for f in range(G):
                pltpu.sync_copy(acc_fs[f], out_hbm.at[_wid(), p * G + f])

    return _sc_agg


_sc_agg2 = _make_sc_agg(2, 2, make_norm=False)


def _tc_h1T(W1, x):
    def body(w_ref, x_ref, h_ref):
        h_ref[...] = lax.dot_general(
            w_ref[...], x_ref[...], (((0,), (1,)), ((), ())),
            preferred_element_type=jnp.float32)

    return pl.pallas_call(
        body, out_shape=jax.ShapeDtypeStruct((16, N), jnp.float32),
    )(W1, x)


def _tc_layer2_prep(agg1p, h1T, dis2d, b1c, W2):
    def body(a_ref, h_ref, d_ref, b_ref, w_ref, o_ref):
        aggsum = jnp.sum(a_ref[...], axis=0)
        dis2 = d_ref[...] * d_ref[...]
        out1 = aggsum + h_ref[...] * dis2 + b_ref[...]
        r = jnp.maximum(out1, 0.0)
        o_ref[...] = lax.dot_general(
            w_ref[...], r, (((0,), (0,)), ((), ())),
            preferred_element_type=jnp.float32)

    return pl.pallas_call(
        body, out_shape=jax.ShapeDtypeStruct((2, N), jnp.float32),
    )(agg1p, h1T, dis2d, b1c, W2)


def _tc_final(agg2p, h2T, dis2d, b2c):
    def body(a_ref, h_ref, d_ref, b_ref, o_ref):
        aggsum = jnp.sum(a_ref[...], axis=0)
        dis2 = d_ref[...] * d_ref[...]
        o2 = aggsum + h_ref[...] * dis2 + b_ref[...]
        m = jnp.max(o2, axis=0, keepdims=True)
        lse = m + jnp.log(jnp.sum(jnp.exp(o2 - m), axis=0, keepdims=True))
        o_ref[...] = o2 - lse

    return pl.pallas_call(
        body, out_shape=jax.ShapeDtypeStruct((2, N), jnp.float32),
    )(agg2p, h2T, dis2d, b2c)


@jax.jit
def kernel(x, edge_index, edge_weight, W1, b1, W2, b2):
    eix = edge_index.astype(jnp.int32).reshape(2 * E)

    h1T = _tc_h1T(W1, x)
    agg1p, norm, disp, _ = _sc_layer1(eix, edge_weight, h1T)
    dis2d = disp[:N].reshape(1, N)
    h2T = _tc_layer2_prep(agg1p, h1T, dis2d, b1.reshape(16, 1), W2)
    agg2p = _sc_agg2(eix, norm, h2T)
    lsmT = _tc_final(agg2p, h2T, dis2d, b2.reshape(2, 1))
    return lsmT.T


# speedup vs baseline: 1.0075x; 1.0075x over previous
import dataclasses
import functools

import jax
import jax.numpy as jnp
from jax import lax
from jax.experimental import pallas as pl
from jax.experimental.pallas import tpu as pltpu
from jax.experimental.pallas import tpu_sc as plsc

N = 10000
E = 320000
NC, NS = 2, 16
NW = NC * NS
EPW = E // NW
L = 16

_mesh = plsc.VectorSubcoreMesh(
    core_axis_name="c", subcore_axis_name="s", num_cores=NC, num_subcores=NS)

_cp = pltpu.CompilerParams()
if "needs_layout_passes" in pltpu.CompilerParams.__dataclass_fields__:
    _cp = dataclasses.replace(_cp, needs_layout_passes=False)


def _wid():
    return lax.axis_index("c") * NS + lax.axis_index("s")


RED = 640
NPAD = NS * RED


def _rsqrt_newton(x):
    i = plsc.bitcast(x, jnp.int32)
    seed = jnp.full((L,), 0x5F3759DF, jnp.int32) - lax.shift_right_logical(
        i, jnp.ones((L,), jnp.int32))
    y = plsc.bitcast(seed, jnp.float32)
    for _ in range(3):
        y = y * (1.5 - 0.5 * x * y * y)
    return y


@functools.partial(
    pl.kernel,
    out_type=(jax.ShapeDtypeStruct((NW, 16, N), jnp.float32),
              jax.ShapeDtypeStruct((E,), jnp.float32),
              jax.ShapeDtypeStruct((NC * NPAD,), jnp.float32),
              jax.ShapeDtypeStruct((NC * NS * N,), jnp.float32)),
    mesh=_mesh,
    compiler_params=_cp,
    name="sc_layer1",
    scratch_types=[
        pltpu.VMEM((N,), jnp.float32),
        pltpu.VMEM((EPW,), jnp.float32),
        pltpu.VMEM((EPW,), jnp.int32),
        pltpu.VMEM((EPW,), jnp.int32),
        pltpu.VMEM((NS * (RED // 2),), jnp.float32),
        pltpu.SemaphoreType.DMA,
    ] + [pltpu.VMEM((N,), jnp.float32)] * 8,
)
def _sc_layer1(eix_hbm, ew_hbm, hT_hbm,
               out_hbm, norm_hbm, dis_hbm, degs_hbm,
               dis_v, nrm_v, src_v, dst_v, red_blk, dma_sem,
               *rows):
    h_fs = rows[:4]
    acc_fs = rows[4:]
    sid = lax.axis_index("s")
    wid = _wid()
    z16 = jnp.zeros((L,), jnp.float32)

    @plsc.parallel_loop(0, N, step=L, unroll=8)
    def _(i):
        dis_v[pl.ds(i, L)] = z16

    for half in range(2):
        dbase = sid * (2 * EPW) + half * EPW
        pltpu.sync_copy(eix_hbm.at[pl.ds(E + dbase, EPW)], src_v)
        pltpu.sync_copy(ew_hbm.at[pl.ds(dbase, EPW)], nrm_v)

        @plsc.parallel_loop(0, EPW, step=L, unroll=4)
        def _(g):
            plsc.addupdate_scatter(
                dis_v, [src_v[pl.ds(g, L)]], nrm_v[pl.ds(g, L)])

    cid = lax.axis_index("c")
    pltpu.sync_copy(dis_v, degs_hbm.at[pl.ds((cid * NS + sid) * N, N)])
    plsc.subcore_barrier()
    tstart = sid * RED
    hw = RED // 2
    for half in range(2):
        copies = [
            pltpu.async_copy(
                degs_hbm.at[pl.ds((cid * NS + j) * N + tstart + half * hw, hw)],
                red_blk.at[pl.ds(j * hw, hw)], dma_sem)
            for j in range(NS)
        ]
        for c in copies:
            c.wait()

        @plsc.parallel_loop(0, hw, step=L, unroll=2)
        def _(k):
            acc = red_blk[pl.ds(k, L)]
            for j in range(1, NS):
                acc = acc + red_blk[pl.ds(j * hw + k, L)]
            dis_v[pl.ds(half * hw + k, L)] = _rsqrt_newton(acc + 1.0)

    pltpu.sync_copy(dis_v.at[pl.ds(0, RED)],
                    dis_hbm.at[pl.ds(cid * NPAD + tstart, RED)])
    plsc.subcore_barrier()
    pltpu.sync_copy(dis_hbm.at[pl.ds(cid * NPAD, N)], dis_v)

    base = wid * EPW
    pltpu.sync_copy(eix_hbm.at[pl.ds(base, EPW)], src_v)
    pltpu.sync_copy(eix_hbm.at[pl.ds(E + base, EPW)], dst_v)
    pltpu.sync_copy(ew_hbm.at[pl.ds(base, EPW)], nrm_v)

    @plsc.parallel_loop(0, EPW, step=L, unroll=8)
    def _(g):
        dsv = plsc.load_gather(dis_v, [src_v[pl.ds(g, L)]])
        ddv = plsc.load_gather(dis_v, [dst_v[pl.ds(g, L)]])
        nrm_v[pl.ds(g, L)] = dsv * nrm_v[pl.ds(g, L)] * ddv

    pltpu.sync_copy(nrm_v, norm_hbm.at[pl.ds(base, EPW)])

    for p in range(4):
        for f in range(4):
            pltpu.sync_copy(hT_hbm.at[p * 4 + f], h_fs[f])

        @plsc.parallel_loop(0, N, step=L, unroll=8)
        def _(i):
            for f in range(4):
                acc_fs[f][pl.ds(i, L)] = z16

        @plsc.parallel_loop(0, EPW, step=L, unroll=8)
        def _(g):
            s16 = src_v[pl.ds(g, L)]
            d16 = dst_v[pl.ds(g, L)]
            n16 = nrm_v[pl.ds(g, L)]
            for f in range(4):
                hf = plsc.load_gather(h_fs[f], [s16])
                plsc.addupdate_scatter(acc_fs[f], [d16], hf * n16)

        for f in range(4):
            pltpu.sync_copy(acc_fs[f], out_hbm.at[wid, p * 4 + f])


@functools.partial(
    pl.kernel,
    out_type=jax.ShapeDtypeStruct((NW, 2, N), jnp.float32),
    mesh=_mesh,
    compiler_params=_cp,
    name="sc_agg2",
    scratch_types=[
        pltpu.VMEM((EPW,), jnp.float32),
        pltpu.VMEM((EPW,), jnp.int32),
        pltpu.VMEM((EPW,), jnp.int32),
    ] + [pltpu.VMEM((N,), jnp.float32)] * 4,
)
def _sc_agg2(eix_hbm, norm_in_hbm, hT_hbm, out_hbm,
             nrm_v, src_v, dst_v, h0, h1, a0, a1):
    h_fs = (h0, h1)
    acc_fs = (a0, a1)
    wid = _wid()
    base = wid * EPW
    pltpu.sync_copy(eix_hbm.at[pl.ds(base, EPW)], src_v)
    pltpu.sync_copy(eix_hbm.at[pl.ds(E + base, EPW)], dst_v)
    pltpu.sync_copy(norm_in_hbm.at[pl.ds(base, EPW)], nrm_v)
    for f in range(2):
        pltpu.sync_copy(hT_hbm.at[f], h_fs[f])

    z16 = jnp.zeros((L,), jnp.float32)

    @plsc.parallel_loop(0, N, step=L, unroll=8)
    def _(i):
        for f in range(2):
            acc_fs[f][pl.ds(i, L)] = z16

    @plsc.parallel_loop(0, EPW, step=L, unroll=8)
    def _(g):
        s16 = src_v[pl.ds(g, L)]
        d16 = dst_v[pl.ds(g, L)]
        n16 = nrm_v[pl.ds(g, L)]
        for f in range(2):
            hf = plsc.load_gather(h_fs[f], [s16])
            plsc.addupdate_scatter(acc_fs[f], [d16], hf * n16)

    for f in range(2):
        pltpu.sync_copy(acc_fs[f], out_hbm.at[wid, f])


def _tc_h1T(W1, x):
    def body(w_ref, x_ref, h_ref):
        h_ref[...] = lax.dot_general(
            w_ref[...], x_ref[...], (((0,), (1,)), ((), ())),
            preferred_element_type=jnp.float32)

    return pl.pallas_call(
        body, out_shape=jax.ShapeDtypeStruct((16, N), jnp.float32),
    )(W1, x)


def _tc_layer2_prep(agg1p, h1T, dis2d, b1c, W2):
    def body(a_ref, h_ref, d_ref, b_ref, w_ref, o_ref):
        aggsum = jnp.sum(a_ref[...], axis=0)
        dis2 = d_ref[...] * d_ref[...]
        out1 = aggsum + h_ref[...] * dis2 + b_ref[...]
        r = jnp.maximum(out1, 0.0)
        o_ref[...] = lax.dot_general(
            w_ref[...], r, (((0,), (0,)), ((), ())),
            preferred_element_type=jnp.float32)

    return pl.pallas_call(
        body, out_shape=jax.ShapeDtypeStruct((2, N), jnp.float32),
    )(agg1p, h1T, dis2d, b1c, W2)


def _tc_final(agg2p, h2T, dis2d, b2c):
    def body(a_ref, h_ref, d_ref, b_ref, o_ref):
        aggsum = jnp.sum(a_ref[...], axis=0)
        dis2 = d_ref[...] * d_ref[...]
        o2 = aggsum + h_ref[...] * dis2 + b_ref[...]
        m = jnp.max(o2, axis=0, keepdims=True)
        lse = m + jnp.log(jnp.sum(jnp.exp(o2 - m), axis=0, keepdims=True))
        o_ref[...] = o2 - lse

    return pl.pallas_call(
        body, out_shape=jax.ShapeDtypeStruct((2, N), jnp.float32),
    )(agg2p, h2T, dis2d, b2c)


@jax.jit
def kernel(x, edge_index, edge_weight, W1, b1, W2, b2):
    eix = edge_index.astype(jnp.int32).reshape(2 * E)

    h1T = _tc_h1T(W1, x)
    agg1p, norm, disp, _ = _sc_layer1(eix, edge_weight, h1T)
    dis2d = disp[:N].reshape(1, N)
    h2T = _tc_layer2_prep(agg1p, h1T, dis2d, b1.reshape(16, 1), W2)
    agg2p = _sc_agg2(eix, norm, h2T)
    lsmT = _tc_final(agg2p, h2T, dis2d, b2.reshape(2, 1))
    return lsmT.T
